# baseline (speedup 1.0000x reference)
import jax
import jax.numpy as jnp
from jax.experimental import pallas as pl
from jax.experimental.pallas import tpu as pltpu

H = 2048; NH = 16; NKV = 16; HD = 128; E = 8; KTOP = 2
I = 1408; IS = 5632; EPS = 1e-6; THETA = 1000000.0; T = 2048

F32 = jnp.float32


def _k1_body(x_ref, ln_ref, w_ref, b_ref, cos_ref, sin_ref, out_ref, scale_ref):
    j = pl.program_id(0)

    @pl.when(j == 0)
    def _():
        xs = x_ref[...]
        scale_ref[...] = jax.lax.rsqrt(
            jnp.mean(xs * xs, axis=-1, keepdims=True) + EPS)

    xn = x_ref[...] * scale_ref[...] * ln_ref[...]
    acc = jax.lax.dot_general(xn, w_ref[...], (((1,), (1,)), ((), ())),
                              preferred_element_type=F32) + b_ref[...]
    c = cos_ref[...]
    s = sin_ref[...]
    x1 = acc[:, :HD // 2]
    x2 = acc[:, HD // 2:]
    roped = jnp.concatenate([x1 * c - x2 * s, x2 * c + x1 * s], axis=-1)
    out_ref[...] = jnp.where(j < 2 * NH, roped, acc)


def _qkv_rope(x, ln1_w, w_qkv, b_qkv, cos, sin):
    nblk = (NH + 2 * NKV)
    return pl.pallas_call(
        _k1_body,
        grid=(nblk,),
        in_specs=[
            pl.BlockSpec((T, H), lambda j: (0, 0)),
            pl.BlockSpec((1, H), lambda j: (0, 0)),
            pl.BlockSpec((HD, H), lambda j: (j, 0)),
            pl.BlockSpec((1, HD), lambda j: (0, j)),
            pl.BlockSpec((T, HD // 2), lambda j: (0, 0)),
            pl.BlockSpec((T, HD // 2), lambda j: (0, 0)),
        ],
        out_specs=pl.BlockSpec((T, HD), lambda j: (0, j)),
        out_shape=jax.ShapeDtypeStruct((T, nblk * HD), F32),
        scratch_shapes=[pltpu.VMEM((T, 1), F32)],
        compiler_params=pltpu.CompilerParams(
            dimension_semantics=("arbitrary",)),
    )(x, ln1_w.reshape(1, H), w_qkv, b_qkv.reshape(1, nblk * HD), cos, sin)


_BQ = 256
_BK = 256


def _k2_body(q_ref, k_ref, v_ref, o_ref):
    i = pl.program_id(1)
    q = q_ref[...] * (HD ** -0.5)

    def body(c, carry):
        acc, m, l = carry
        kc = k_ref[pl.ds(c * _BK, _BK), :]
        vc = v_ref[pl.ds(c * _BK, _BK), :]
        s = jax.lax.dot_general(q, kc, (((1,), (1,)), ((), ())),
                                preferred_element_type=F32)
        rows = i * _BQ + jax.lax.broadcasted_iota(jnp.int32, (_BQ, _BK), 0)
        cols = c * _BK + jax.lax.broadcasted_iota(jnp.int32, (_BQ, _BK), 1)
        s = jnp.where(rows >= cols, s, -1e9)
        m_new = jnp.maximum(m, jnp.max(s, axis=-1, keepdims=True))
        p = jnp.exp(s - m_new)
        alpha = jnp.exp(m - m_new)
        l_new = l * alpha + jnp.sum(p, axis=-1, keepdims=True)
        acc_new = acc * alpha + jnp.dot(p, vc, preferred_element_type=F32)
        return acc_new, m_new, l_new

    init = (jnp.zeros((_BQ, HD), F32),
            jnp.full((_BQ, 1), -1e30, F32),
            jnp.zeros((_BQ, 1), F32))
    acc, m, l = jax.lax.fori_loop(0, i + 1, body, init)
    o_ref[...] = acc / l


def _attention(qkv):
    return pl.pallas_call(
        _k2_body,
        grid=(NH, T // _BQ),
        in_specs=[
            pl.BlockSpec((_BQ, HD), lambda h, i: (i, h)),
            pl.BlockSpec((T, HD), lambda h, i: (0, NH + h)),
            pl.BlockSpec((T, HD), lambda h, i: (0, 2 * NH + h)),
        ],
        out_specs=pl.BlockSpec((_BQ, HD), lambda h, i: (i, h)),
        out_shape=jax.ShapeDtypeStruct((T, NH * HD), F32),
        compiler_params=pltpu.CompilerParams(
            dimension_semantics=("parallel", "arbitrary")),
    )(qkv, qkv, qkv)


_BM3 = 256


def _k3_body(x_ref, o_ref, wo_ref, ln2_ref, wr_ref,
             x1_ref, xn2_ref, logits_ref):
    x1 = x_ref[...] + jax.lax.dot_general(
        o_ref[...], wo_ref[...], (((1,), (1,)), ((), ())),
        preferred_element_type=F32)
    scale = jax.lax.rsqrt(jnp.mean(x1 * x1, axis=-1, keepdims=True) + EPS)
    xn2 = x1 * scale * ln2_ref[...]
    x1_ref[...] = x1
    xn2_ref[...] = xn2
    logits_ref[...] = jax.lax.dot_general(
        xn2, wr_ref[...], (((1,), (1,)), ((), ())),
        preferred_element_type=F32)


def _oproj_norm_router(x, o, w_o, ln2_w, wr):
    return pl.pallas_call(
        _k3_body,
        grid=(T // _BM3,),
        in_specs=[
            pl.BlockSpec((_BM3, H), lambda i: (i, 0)),
            pl.BlockSpec((_BM3, NH * HD), lambda i: (i, 0)),
            pl.BlockSpec((H, NH * HD), lambda i: (0, 0)),
            pl.BlockSpec((1, H), lambda i: (0, 0)),
            pl.BlockSpec((128, H), lambda i: (0, 0)),
        ],
        out_specs=[
            pl.BlockSpec((_BM3, H), lambda i: (i, 0)),
            pl.BlockSpec((_BM3, H), lambda i: (i, 0)),
            pl.BlockSpec((_BM3, 128), lambda i: (i, 0)),
        ],
        out_shape=[
            jax.ShapeDtypeStruct((T, H), F32),
            jax.ShapeDtypeStruct((T, H), F32),
            jax.ShapeDtypeStruct((T, 128), F32),
        ],
        compiler_params=pltpu.CompilerParams(
            dimension_semantics=("arbitrary",)),
    )(x, o, w_o, ln2_w.reshape(1, H), wr)


_BN4 = 128


def _k4_body(xn_ref, wg_ref, wu_ref, act_ref):
    xn = xn_ref[...]
    g = jax.lax.dot_general(xn, wg_ref[...], (((1,), (1,)), ((), ())),
                            preferred_element_type=F32)
    u = jax.lax.dot_general(xn, wu_ref[...], (((1,), (1,)), ((), ())),
                            preferred_element_type=F32)
    act_ref[...] = (g * jax.nn.sigmoid(g)) * u


def _shared_gateup(xn2, w_gu_shared):
    return pl.pallas_call(
        _k4_body,
        grid=(IS // _BN4,),
        in_specs=[
            pl.BlockSpec((T, H), lambda j: (0, 0)),
            pl.BlockSpec((_BN4, H), lambda j: (j, 0)),
            pl.BlockSpec((_BN4, H), lambda j: (IS // _BN4 + j, 0)),
        ],
        out_specs=pl.BlockSpec((T, _BN4), lambda j: (0, j)),
        out_shape=jax.ShapeDtypeStruct((T, IS), F32),
        compiler_params=pltpu.CompilerParams(
            dimension_semantics=("arbitrary",)),
    )(xn2, w_gu_shared, w_gu_shared)


_BI = 128


def _k6_body(xn_ref, wg_ref, wu_ref, wd_ref, wt_ref, out_ref):
    e = pl.program_id(0)
    i = pl.program_id(1)
    xn = xn_ref[...]
    g = jax.lax.dot_general(xn, wg_ref[0], (((1,), (1,)), ((), ())),
                            preferred_element_type=F32)
    u = jax.lax.dot_general(xn, wu_ref[0], (((1,), (1,)), ((), ())),
                            preferred_element_type=F32)
    act = (g * jax.nn.sigmoid(g)) * u * wt_ref[0, 0][:, None]
    part = jax.lax.dot_general(act, wd_ref[0], (((1,), (1,)), ((), ())),
                               preferred_element_type=F32)

    @pl.when(jnp.logical_and(e == 0, i == 0))
    def _():
        out_ref[...] = part

    @pl.when(jnp.logical_or(e != 0, i != 0))
    def _():
        out_ref[...] += part


def _moe_dense(xn2, w_gu_exp, w_down_exp, wts):
    return pl.pallas_call(
        _k6_body,
        grid=(E, I // _BI),
        in_specs=[
            pl.BlockSpec((T, H), lambda e, i: (0, 0)),
            pl.BlockSpec((1, _BI, H), lambda e, i: (e, i, 0)),
            pl.BlockSpec((1, _BI, H), lambda e, i: (e, I // _BI + i, 0)),
            pl.BlockSpec((1, H, _BI), lambda e, i: (e, 0, i)),
            pl.BlockSpec((1, 1, T), lambda e, i: (e, 0, 0)),
        ],
        out_specs=pl.BlockSpec((T, H), lambda e, i: (0, 0)),
        out_shape=jax.ShapeDtypeStruct((T, H), F32),
        compiler_params=pltpu.CompilerParams(
            dimension_semantics=("arbitrary", "arbitrary")),
    )(xn2, w_gu_exp, w_gu_exp, w_down_exp, wts)


_BM5 = 256
_BN5 = 256


def _k5_body(act_ref, wd_ref, x1_ref, logits_ref, fused_ref, out_ref):
    shared = jax.lax.dot_general(act_ref[...], wd_ref[...],
                                 (((1,), (1,)), ((), ())),
                                 preferred_element_type=F32)
    sg = jax.nn.sigmoid(logits_ref[...][:, E:E + 1])
    out_ref[...] = x1_ref[...] + sg * shared + fused_ref[...]


def _shared_down_combine(act_s, w_down_shared, x1, logits, fused):
    return pl.pallas_call(
        _k5_body,
        grid=(T // _BM5, H // _BN5),
        in_specs=[
            pl.BlockSpec((_BM5, IS), lambda i, j: (i, 0)),
            pl.BlockSpec((_BN5, IS), lambda i, j: (j, 0)),
            pl.BlockSpec((_BM5, _BN5), lambda i, j: (i, j)),
            pl.BlockSpec((_BM5, 128), lambda i, j: (i, 0)),
            pl.BlockSpec((_BM5, _BN5), lambda i, j: (i, j)),
        ],
        out_specs=pl.BlockSpec((_BM5, _BN5), lambda i, j: (i, j)),
        out_shape=jax.ShapeDtypeStruct((T, H), F32),
        compiler_params=pltpu.CompilerParams(
            dimension_semantics=("parallel", "arbitrary")),
    )(act_s, w_down_shared, x1, logits, fused)


def kernel(positions, x, ln1_w, ln2_w, w_qkv, b_qkv, w_o, w_gate, w_sgate,
           w_gu_shared, w_down_shared, w_gu_exp, w_down_exp):
    half = HD // 2
    inv = THETA ** (-jnp.arange(half, dtype=F32) / half)
    f = positions.astype(F32)[:, None] * inv
    cos = jnp.cos(f)
    sin = jnp.sin(f)

    qkv = _qkv_rope(x, ln1_w, w_qkv, b_qkv, cos, sin)
    o = _attention(qkv)

    wr = jnp.concatenate(
        [w_gate, w_sgate, jnp.zeros((128 - E - 1, H), F32)], axis=0)
    x1, xn2, logits = _oproj_norm_router(x, o, w_o, ln2_w, wr)

    probs = jax.nn.softmax(logits[:, :E], axis=-1)
    vals, idx = jax.lax.top_k(probs, KTOP)
    vals = vals / jnp.sum(vals, axis=-1, keepdims=True)
    w_dense = jnp.zeros((T, E), F32).at[jnp.arange(T)[:, None], idx].set(vals)
    wts = w_dense.T.reshape(E, 1, T)

    fused = _moe_dense(xn2, w_gu_exp, w_down_exp, wts)
    act_s = _shared_gateup(xn2, w_gu_shared)
    return _shared_down_combine(act_s, w_down_shared, x1, logits, fused)
